# Initial kernel scaffold; baseline (speedup 1.0000x reference)
import functools

import jax
import jax.numpy as jnp
from jax import lax
from jax.experimental import pallas as pl
from jax.experimental.pallas import tpu as pltpu
from jax.experimental.pallas import tpu_sc as plsc

VOCAB = 1000000
DIM = 32
B = 16384
L = 50
N = B * L

_info = plsc.get_sparse_core_info()
NC, NS = _info.num_cores, _info.num_subcores
NW = NC * NS
N_PER_W = N // NW
CHUNK = 1024
N_CHUNKS = N_PER_W // CHUNK


def _gather_body(x_hbm, emb_hbm, out_hbm, idx_v, rows_v, sem):
    wid = lax.axis_index("s") * NC + lax.axis_index("c")
    base = wid * N_PER_W
    pltpu.sync_copy(x_hbm.at[pl.ds(base, N_PER_W)], idx_v)

    def chunk(i, _):
        off = i * CHUNK
        pltpu.async_copy(
            emb_hbm.at[idx_v.at[pl.ds(off, CHUNK)]], rows_v, sem
        ).wait()
        pltpu.sync_copy(rows_v, out_hbm.at[pl.ds(base + off, CHUNK)])
        return 0

    lax.fori_loop(0, N_CHUNKS, chunk, 0)


@jax.jit
def _lookup(x_flat, emb):
    mesh = plsc.VectorSubcoreMesh(core_axis_name="c", subcore_axis_name="s")
    return pl.kernel(
        _gather_body,
        mesh=mesh,
        out_type=jax.ShapeDtypeStruct((N, DIM), jnp.float32),
        scratch_types=[
            pltpu.VMEM((N_PER_W,), jnp.int32),
            pltpu.VMEM((CHUNK, DIM), jnp.float32),
            pltpu.SemaphoreType.DMA,
        ],
    )(x_flat, emb)


def kernel(x, emb):
    out = _lookup(x.reshape(N).astype(jnp.int32), emb)
    return out.reshape(B, L, DIM)


# speedup vs baseline: 1.1031x; 1.1031x over previous
import functools

import jax
import jax.numpy as jnp
from jax import lax
from jax.experimental import pallas as pl
from jax.experimental.pallas import tpu as pltpu
from jax.experimental.pallas import tpu_sc as plsc

VOCAB = 1000000
DIM = 32
B = 16384
L = 50
N = B * L

_info = plsc.get_sparse_core_info()
NC, NS = _info.num_cores, _info.num_subcores
NW = NC * NS
N_PER_W = N // NW
CHUNK = 1024
N_CHUNKS = N_PER_W // CHUNK


def _gather_body(x_hbm, emb_hbm, out_hbm, idx_v, rows_v, sem):
    wid = lax.axis_index("s") * NC + lax.axis_index("c")
    base = wid * N_PER_W
    pltpu.sync_copy(x_hbm.at[pl.ds(base, N_PER_W)], idx_v)

    def chunk(i, _):
        off = i * CHUNK
        pltpu.async_copy(
            emb_hbm.at[idx_v.at[pl.ds(off, CHUNK)]], rows_v, sem
        ).wait()
        pltpu.sync_copy(rows_v, out_hbm.at[pl.ds(base + off, CHUNK)])
        return 0

    lax.fori_loop(0, N_CHUNKS, chunk, 0)


@jax.jit
def _lookup(x_flat, emb):
    mesh = plsc.VectorSubcoreMesh(core_axis_name="c", subcore_axis_name="s")
    return pl.kernel(
        _gather_body,
        mesh=mesh,
        compiler_params=pltpu.CompilerParams(use_tc_tiling_on_sc=False),
        out_type=jax.ShapeDtypeStruct((N, DIM), jnp.float32),
        scratch_types=[
            pltpu.VMEM((N_PER_W,), jnp.int32),
            pltpu.VMEM((CHUNK, DIM), jnp.float32),
            pltpu.SemaphoreType.DMA,
        ],
    )(x_flat, emb)


def kernel(x, emb):
    out = _lookup(x.reshape(N).astype(jnp.int32), emb)
    return out.reshape(B, L, DIM)


# speedup vs baseline: 1.1133x; 1.0092x over previous
import jax
import jax.numpy as jnp
from jax import lax
from jax.experimental import pallas as pl
from jax.experimental.pallas import tpu as pltpu
from jax.experimental.pallas import tpu_sc as plsc

VOCAB = 1000000
DIM = 32
B = 16384
L = 50
N = B * L

_info = plsc.get_sparse_core_info()
NC, NS = _info.num_cores, _info.num_subcores
NW = NC * NS
N_PER_W = N // NW
CHUNK = 640
N_CHUNKS = N_PER_W // CHUNK
NBUF = 4
DEPTH = 2


def _gather_body(x_hbm, emb_hbm, out_hbm, idx_v,
                 r0, r1, r2, r3, g0, g1, g2, g3, w0, w1, w2, w3):
    rows = (r0, r1, r2, r3)
    gsem = (g0, g1, g2, g3)
    wsem = (w0, w1, w2, w3)
    wid = lax.axis_index("s") * NC + lax.axis_index("c")
    base = wid * N_PER_W
    pltpu.sync_copy(x_hbm.at[pl.ds(base, N_PER_W)], idx_v)

    def idx_slice(j):
        return idx_v.at[pl.ds(pl.multiple_of(j * CHUNK, 8), CHUNK)]

    def out_slice(j):
        return out_hbm.at[pl.ds(pl.multiple_of(base + j * CHUNK, 8), CHUNK)]

    def g_start(j, b):
        pltpu.async_copy(emb_hbm.at[idx_slice(j)], rows[b], gsem[b])

    def g_wait(b):
        pltpu.make_async_copy(emb_hbm.at[idx_slice(0)], rows[b], gsem[b]).wait()

    def w_start(j, b):
        pltpu.async_copy(rows[b], out_slice(j), wsem[b])

    def w_wait(b):
        pltpu.make_async_copy(rows[b], out_slice(0), wsem[b]).wait()

    g_start(0, 0)
    g_start(1, 1)
    for j in (0, 1):
        g_wait(j)
        w_start(j, j)
        g_start(j + DEPTH, j + DEPTH)

    def group(g, carry):
        j0 = NBUF * g + DEPTH
        for b0 in range(NBUF):
            j = j0 + b0
            b = (DEPTH + b0) % NBUF
            g_wait(b)
            w_start(j, b)
            w_wait(b0)
            g_start(j + DEPTH, b0)
        return carry

    lax.fori_loop(0, (N_CHUNKS - NBUF) // NBUF, group, 0)

    for k in (DEPTH, DEPTH - 1):
        j = N_CHUNKS - k
        b = j % NBUF
        g_wait(b)
        w_start(j, b)
    for b in range(NBUF):
        w_wait(b)


@jax.jit
def _lookup(x_flat, emb):
    mesh = plsc.VectorSubcoreMesh(core_axis_name="c", subcore_axis_name="s")
    return pl.kernel(
        _gather_body,
        mesh=mesh,
        compiler_params=pltpu.CompilerParams(use_tc_tiling_on_sc=False),
        out_type=jax.ShapeDtypeStruct((N, DIM), jnp.float32),
        scratch_types=(
            [pltpu.VMEM((N_PER_W,), jnp.int32)]
            + [pltpu.VMEM((CHUNK, DIM), jnp.float32) for _ in range(NBUF)]
            + [pltpu.SemaphoreType.DMA for _ in range(2 * NBUF)]
        ),
    )(x_flat, emb)


def kernel(x, emb):
    out = _lookup(x.reshape(N).astype(jnp.int32), emb)
    return out.reshape(B, L, DIM)


# speedup vs baseline: 1.5023x; 1.3495x over previous
import jax
import jax.numpy as jnp
from jax import lax
from jax.experimental import pallas as pl
from jax.experimental.pallas import tpu as pltpu
from jax.experimental.pallas import tpu_sc as plsc

VOCAB = 1000000
DIM = 32
B = 16384
L = 50

_info = plsc.get_sparse_core_info()
NC, NS = _info.num_cores, _info.num_subcores
NW = NC * NS
B_PER_W = B // NW
SUB = 256
H = B_PER_W // SUB
PACK = 4
PROWS = VOCAB // PACK


def _body(xt_hbm, embp_hbm, out_hbm,
          idx_raw, gidx0, gidx1, off0, off1, p0, p1, tb, gs0, gs1):
    gidx = (gidx0, gidx1)
    offb = (off0, off1)
    pbuf = (p0, p1)
    gsem = (gs0, gs1)
    wid = lax.axis_index("s") * NC + lax.axis_index("c")
    bbase = wid * B_PER_W
    lanes = lax.iota(jnp.int32, 16)

    def prep(l, h, pb):
        c = bbase + h * SUB
        pltpu.sync_copy(xt_hbm.at[l, pl.ds(c, SUB)], idx_raw)

        def grp(g, carry):
            s = pl.ds(pl.multiple_of(g * 16, 8), 16)
            v = idx_raw[s]
            gidx[pb][s] = jax.lax.shift_right_logical(v, 2)
            offb[pb][s] = jax.lax.shift_left(jnp.bitwise_and(v, 3), 5)
            return carry

        lax.fori_loop(0, SUB // 16, grp, 0)

    def g_start(pb):
        pltpu.async_copy(embp_hbm.at[gidx[pb]], pbuf[pb], gsem[pb])

    def g_wait(pb):
        pltpu.make_async_copy(embp_hbm.at[gidx[pb]], pbuf[pb], gsem[pb]).wait()

    def extract(pb):
        def grp(g, carry):
            s = pl.ds(pl.multiple_of(g * 16, 8), 16)
            rows = g * 16 + lanes
            offv = offb[pb][s]

            def feat(f, carry2):
                t = plsc.load_gather(pbuf[pb], [rows, offv + f])
                tb[f, s] = t
                return carry2

            lax.fori_loop(0, DIM, feat, 0)
            return carry

        lax.fori_loop(0, SUB // 16, grp, 0)

    def wb(l, h):
        pltpu.sync_copy(tb, out_hbm.at[l, :, pl.ds(bbase + h * SUB, SUB)])

    prep(0, 0, 0)
    g_start(0)

    def lbody(l, carry):
        prep(l, 1, 1)
        g_start(1)
        g_wait(0)
        extract(0)
        wb(l, 0)

        @pl.when(l + 1 < L)
        def _():
            prep(l + 1, 0, 0)
            g_start(0)

        g_wait(1)
        extract(1)
        wb(l, 1)
        return carry

    lax.fori_loop(0, L, lbody, 0)


@jax.jit
def _lookup(xt, embp):
    mesh = plsc.VectorSubcoreMesh(core_axis_name="c", subcore_axis_name="s")
    return pl.kernel(
        _body,
        mesh=mesh,
        compiler_params=pltpu.CompilerParams(needs_layout_passes=False),
        out_type=jax.ShapeDtypeStruct((L, DIM, B), jnp.float32),
        scratch_types=(
            [pltpu.VMEM((SUB,), jnp.int32) for _ in range(5)]
            + [pltpu.VMEM((SUB, 128), jnp.float32) for _ in range(2)]
            + [pltpu.VMEM((DIM, SUB), jnp.float32)]
            + [pltpu.SemaphoreType.DMA for _ in range(2)]
        ),
    )(xt, embp)


def kernel(x, emb):
    outt = _lookup(x.T, emb.reshape(PROWS, PACK * DIM))
    return jnp.transpose(outt, (2, 0, 1))


# speedup vs baseline: 1.6152x; 1.0752x over previous
import jax
import jax.numpy as jnp
from jax import lax
from jax.experimental import pallas as pl
from jax.experimental.pallas import tpu as pltpu
from jax.experimental.pallas import tpu_sc as plsc

VOCAB = 1000000
DIM = 32
B = 16384
L = 50

_info = plsc.get_sparse_core_info()
NC, NS = _info.num_cores, _info.num_subcores
NW = NC * NS
B_PER_W = B // NW
SUB = 256
H = B_PER_W // SUB
PACK = 4
PROWS = VOCAB // PACK


def _body(xt_hbm, embp_hbm, out_hbm,
          idx_all, gidx0, gidx1, off0, off1, p0, p1, tb0, tb1,
          gs0, gs1, ws0, ws1):
    gidx = (gidx0, gidx1)
    offb = (off0, off1)
    pbuf = (p0, p1)
    tb = (tb0, tb1)
    gsem = (gs0, gs1)
    wsem = (ws0, ws1)
    wid = lax.axis_index("s") * NC + lax.axis_index("c")
    bbase = wid * B_PER_W
    lanes = lax.iota(jnp.int32, 16)

    pltpu.sync_copy(xt_hbm.at[:, pl.ds(bbase, B_PER_W)], idx_all)

    def prep(l, h, pb):
        def grp(g, carry):
            s = pl.ds(pl.multiple_of(g * 16, 8), 16)
            v = idx_all[l, pl.ds(pl.multiple_of(h * SUB + g * 16, 8), 16)]
            gidx[pb][s] = jax.lax.shift_right_logical(v, 2)
            offb[pb][s] = jax.lax.shift_left(jnp.bitwise_and(v, 3), 5)
            return carry

        lax.fori_loop(0, SUB // 16, grp, 0)

    def g_start(pb):
        pltpu.async_copy(embp_hbm.at[gidx[pb]], pbuf[pb], gsem[pb])

    def g_wait(pb):
        pltpu.make_async_copy(embp_hbm.at[gidx[pb]], pbuf[pb], gsem[pb]).wait()

    def extract(pb):
        def grp(g, carry):
            s = pl.ds(pl.multiple_of(g * 16, 8), 16)
            rows = g * 16 + lanes
            offv = offb[pb][s]

            def feat(f, carry2):
                t = plsc.load_gather(pbuf[pb], [rows, offv + f])
                tb[pb][f, s] = t
                return carry2

            lax.fori_loop(0, DIM, feat, 0)
            return carry

        lax.fori_loop(0, SUB // 16, grp, 0)

    def wb_start(l, h):
        pltpu.async_copy(
            tb[h], out_hbm.at[l, :, pl.ds(bbase + h * SUB, SUB)], wsem[h]
        )

    def wb_wait(pb):
        pltpu.make_async_copy(
            tb[pb], out_hbm.at[0, :, pl.ds(bbase, SUB)], wsem[pb]
        ).wait()

    prep(0, 0, 0)
    g_start(0)

    def lbody(l, carry):
        prep(l, 1, 1)
        g_start(1)
        g_wait(0)

        @pl.when(l > 0)
        def _():
            wb_wait(0)

        extract(0)
        wb_start(l, 0)

        @pl.when(l + 1 < L)
        def _():
            prep(l + 1, 0, 0)
            g_start(0)

        g_wait(1)

        @pl.when(l > 0)
        def _():
            wb_wait(1)

        extract(1)
        wb_start(l, 1)
        return carry

    lax.fori_loop(0, L, lbody, 0)
    wb_wait(0)
    wb_wait(1)


@jax.jit
def _lookup(xt, embp):
    mesh = plsc.VectorSubcoreMesh(core_axis_name="c", subcore_axis_name="s")
    return pl.kernel(
        _body,
        mesh=mesh,
        compiler_params=pltpu.CompilerParams(needs_layout_passes=False),
        out_type=jax.ShapeDtypeStruct((L, DIM, B), jnp.float32),
        scratch_types=(
            [pltpu.VMEM((L, B_PER_W), jnp.int32)]
            + [pltpu.VMEM((SUB,), jnp.int32) for _ in range(4)]
            + [pltpu.VMEM((SUB, 128), jnp.float32) for _ in range(2)]
            + [pltpu.VMEM((DIM, SUB), jnp.float32) for _ in range(2)]
            + [pltpu.SemaphoreType.DMA for _ in range(4)]
        ),
    )(xt, embp)


def kernel(x, emb):
    outt = _lookup(x.T, emb.reshape(PROWS, PACK * DIM))
    return jnp.transpose(outt, (2, 0, 1))


# speedup vs baseline: 1.6176x; 1.0015x over previous
import jax
import jax.numpy as jnp
from jax import lax
from jax.experimental import pallas as pl
from jax.experimental.pallas import tpu as pltpu
from jax.experimental.pallas import tpu_sc as plsc

VOCAB = 1000000
DIM = 32
B = 16384
L = 50

_info = plsc.get_sparse_core_info()
NC, NS = _info.num_cores, _info.num_subcores
NW = NC * NS
B_PER_W = B // NW
SUB = 256
H = B_PER_W // SUB
PACK = 4
PROWS = VOCAB // PACK


def _body(xt_hbm, embp_hbm, out_hbm,
          idx_all, gidx0, gidx1, off0, off1, p0, p1, tb0, tb1,
          gs0, gs1, ws0, ws1):
    gidx = (gidx0, gidx1)
    offb = (off0, off1)
    pbuf = (p0, p1)
    tb = (tb0, tb1)
    gsem = (gs0, gs1)
    wsem = (ws0, ws1)
    wid = lax.axis_index("s") * NC + lax.axis_index("c")
    bbase = wid * B_PER_W
    lanes = lax.iota(jnp.int32, 16)

    pltpu.sync_copy(xt_hbm.at[:, pl.ds(bbase, B_PER_W)], idx_all)

    def prep(l, h, pb):
        def grp(g, carry):
            s = pl.ds(pl.multiple_of(g * 16, 8), 16)
            v = idx_all[l, pl.ds(pl.multiple_of(h * SUB + g * 16, 8), 16)]
            gidx[pb][s] = jax.lax.shift_right_logical(v, 2)
            offb[pb][s] = jax.lax.shift_left(jnp.bitwise_and(v, 3), 5)
            return carry

        lax.fori_loop(0, SUB // 16, grp, 0)

    def g_start(pb):
        pltpu.async_copy(embp_hbm.at[gidx[pb]], pbuf[pb], gsem[pb])

    def g_wait(pb):
        pltpu.make_async_copy(embp_hbm.at[gidx[pb]], pbuf[pb], gsem[pb]).wait()

    def extract(pb):
        def grp(g, carry):
            s = pl.ds(pl.multiple_of(g * 16, 8), 16)
            rows = g * 16 + lanes
            offv = offb[pb][s]

            for f in range(DIM):
                tb[pb][f, s] = plsc.load_gather(pbuf[pb], [rows, offv + f])
            return carry

        lax.fori_loop(0, SUB // 16, grp, 0)

    def wb_start(l, h):
        pltpu.async_copy(
            tb[h], out_hbm.at[l, :, pl.ds(bbase + h * SUB, SUB)], wsem[h]
        )

    def wb_wait(pb):
        pltpu.make_async_copy(
            tb[pb], out_hbm.at[0, :, pl.ds(bbase, SUB)], wsem[pb]
        ).wait()

    prep(0, 0, 0)
    g_start(0)

    def lbody(l, carry):
        prep(l, 1, 1)
        g_start(1)
        g_wait(0)

        @pl.when(l > 0)
        def _():
            wb_wait(0)

        extract(0)
        wb_start(l, 0)

        @pl.when(l + 1 < L)
        def _():
            prep(l + 1, 0, 0)
            g_start(0)

        g_wait(1)

        @pl.when(l > 0)
        def _():
            wb_wait(1)

        extract(1)
        wb_start(l, 1)
        return carry

    lax.fori_loop(0, L, lbody, 0)
    wb_wait(0)
    wb_wait(1)


@jax.jit
def _lookup(xt, embp):
    mesh = plsc.VectorSubcoreMesh(core_axis_name="c", subcore_axis_name="s")
    return pl.kernel(
        _body,
        mesh=mesh,
        compiler_params=pltpu.CompilerParams(needs_layout_passes=False),
        out_type=jax.ShapeDtypeStruct((L, DIM, B), jnp.float32),
        scratch_types=(
            [pltpu.VMEM((L, B_PER_W), jnp.int32)]
            + [pltpu.VMEM((SUB,), jnp.int32) for _ in range(4)]
            + [pltpu.VMEM((SUB, 128), jnp.float32) for _ in range(2)]
            + [pltpu.VMEM((DIM, SUB), jnp.float32) for _ in range(2)]
            + [pltpu.SemaphoreType.DMA for _ in range(4)]
        ),
    )(xt, embp)


def kernel(x, emb):
    outt = _lookup(x.T, emb.reshape(PROWS, PACK * DIM))
    return jnp.transpose(outt, (2, 0, 1))


# speedup vs baseline: 1.6187x; 1.0007x over previous
import jax
import jax.numpy as jnp
from jax import lax
from jax.experimental import pallas as pl
from jax.experimental.pallas import tpu as pltpu
from jax.experimental.pallas import tpu_sc as plsc

VOCAB = 1000000
DIM = 32
B = 16384
L = 50

_info = plsc.get_sparse_core_info()
NC, NS = _info.num_cores, _info.num_subcores
NW = NC * NS
B_PER_W = B // NW
SUB = 256
PACK = 4
PROWS = VOCAB // PACK


def _body(xt_hbm, embp_hbm, out_hbm,
          idx_all, gidx0, gidx1, off0, off1, p0, p1, tb0, tb1,
          gs0, gs1, ws0, ws1):
    gidx = (gidx0, gidx1)
    offb = (off0, off1)
    pbuf = (p0, p1)
    tb = (tb0, tb1)
    gsem = (gs0, gs1)
    wsem = (ws0, ws1)
    wid = lax.axis_index("s") * NC + lax.axis_index("c")
    bbase = wid * B_PER_W
    lanes = lax.iota(jnp.int32, 16)

    pltpu.sync_copy(xt_hbm.at[:, pl.ds(bbase, B_PER_W)], idx_all)

    def prep(l, h, pb):
        def grp(g, carry):
            s = pl.ds(pl.multiple_of(g * 16, 8), 16)
            v = idx_all[l, pl.ds(pl.multiple_of(h * SUB + g * 16, 8), 16)]
            gidx[pb][s] = jax.lax.shift_right_logical(v, 2)
            offb[pb][s] = jax.lax.shift_left(jnp.bitwise_and(v, 3), 5)
            return carry

        lax.fori_loop(0, SUB // 16, grp, 0)

    def g_start(pb):
        pltpu.async_copy(embp_hbm.at[gidx[pb]], pbuf[pb], gsem[pb])

    def g_wait(pb):
        pltpu.make_async_copy(embp_hbm.at[gidx[pb]], pbuf[pb], gsem[pb]).wait()

    def extract(pb):
        def grp(g, carry):
            s = pl.ds(pl.multiple_of(g * 16, 8), 16)
            rows = g * 16 + lanes
            offv = offb[pb][s]

            for f in range(DIM):
                tb[pb][f, s] = plsc.load_gather(pbuf[pb], [rows, offv + f])
            return carry

        lax.fori_loop(0, SUB // 16, grp, 0)

    def wb_start(l, h):
        pltpu.async_copy(
            tb[h], out_hbm.at[l, :, pl.ds(bbase + h * SUB, SUB)], wsem[h]
        )

    def wb_wait(pb):
        pltpu.make_async_copy(
            tb[pb], out_hbm.at[0, :, pl.ds(bbase, SUB)], wsem[pb]
        ).wait()

    prep(0, 0, 0)
    g_start(0)

    def lbody(l, carry):
        prep(l, 1, 1)
        g_start(1)
        g_wait(0)

        @pl.when(l > 0)
        def _():
            wb_wait(0)

        extract(0)
        wb_start(l, 0)

        @pl.when(l + 1 < L)
        def _():
            prep(l + 1, 0, 0)
            g_start(0)

        g_wait(1)

        @pl.when(l > 0)
        def _():
            wb_wait(1)

        extract(1)
        wb_start(l, 1)
        return carry

    lax.fori_loop(0, L, lbody, 0)
    wb_wait(0)
    wb_wait(1)


@jax.jit
def _lookup(xt, embp):
    mesh = plsc.VectorSubcoreMesh(core_axis_name="c", subcore_axis_name="s")
    return pl.kernel(
        _body,
        mesh=mesh,
        compiler_params=pltpu.CompilerParams(needs_layout_passes=False),
        out_type=jax.ShapeDtypeStruct((L, DIM, B), jnp.float32),
        scratch_types=(
            [pltpu.VMEM((L, B_PER_W), jnp.int32)]
            + [pltpu.VMEM((SUB,), jnp.int32) for _ in range(4)]
            + [pltpu.VMEM((SUB, 128), jnp.float32) for _ in range(2)]
            + [pltpu.VMEM((DIM, SUB), jnp.float32) for _ in range(2)]
            + [pltpu.SemaphoreType.DMA for _ in range(4)]
        ),
    )(xt, embp)


def kernel(x, emb):
    outt = _lookup(x.T, emb.reshape(PROWS, PACK * DIM))
    return jnp.transpose(outt, (2, 0, 1))
